# Initial kernel scaffold; baseline (speedup 1.0000x reference)
import jax
import jax.numpy as jnp
from jax.experimental import pallas as pl


def kernel(x_cell, x_io, edge_index_cc, edge_index_cio, edge_index_ioc, W_cc, a_src_cc, a_dst_cc, b_cc, W_cio, a_src_cio, a_dst_cio, b_cio, W_ioc, a_src_ioc, a_dst_ioc, b_ioc, Wt_cell, bt_cell, g_cell, beta_cell, Wt_io, bt_io, g_io, beta_io):
    raise NotImplementedError("write your pallas kernel here")


# speedup vs baseline: 4.4162x; 4.4162x over previous
import functools

import jax
import jax.numpy as jnp
from jax import lax
from jax.experimental import pallas as pl
from jax.experimental.pallas import tpu as pltpu
from jax.experimental.pallas import tpu_sc as plsc

HD = 128
H = 4
C = 32
NCELL = 50000
NIO = 10000

NC = 2
NS = 16
NW = NC * NS

BM = 400
NCELL_P = 50400
NIO_P = 10400

BSZ = 1024
NBLK_CC = 16
EP_CC = NW * NBLK_CC * BSZ
NBLK_IO = 4
EP_IO = NW * NBLK_IO * BSZ

CH = 4096
NCH_CELL = 13
NUMP_CELL = NCH_CELL * CH
NCH_IO = 3
NUMP_IO = NCH_IO * CH
ACC_R = 4352
JUNK = ACC_R - 1
CLEN = BSZ + 256
TRASH = BSZ + 192


def _prep_cell_body(x, wcc, wcio, wioc, accs, accd, acios, aiocd,
                    hcc, hcio, alccs, alccd, alcios, aliocd):
    xv = x[...]
    h1 = jnp.dot(xv, wcc[...], preferred_element_type=jnp.float32)
    h2 = jnp.dot(xv, wcio[...], preferred_element_type=jnp.float32)
    h3 = jnp.dot(xv, wioc[...], preferred_element_type=jnp.float32)
    hcc[...] = h1
    hcio[...] = h2
    alccs[...] = jnp.dot(h1, accs[...], preferred_element_type=jnp.float32)
    alccd[...] = jnp.dot(h1, accd[...], preferred_element_type=jnp.float32)
    alcios[...] = jnp.dot(h2, acios[...], preferred_element_type=jnp.float32)
    aliocd[...] = jnp.dot(h3, aiocd[...], preferred_element_type=jnp.float32)


def _prep_io_body(x, wioc, wcio, aiocs, aciod, hioc, aliocs, alciod):
    xv = x[...]
    h1 = jnp.dot(xv, wioc[...], preferred_element_type=jnp.float32)
    h2 = jnp.dot(xv, wcio[...], preferred_element_type=jnp.float32)
    hioc[...] = h1
    aliocs[...] = jnp.dot(h1, aiocs[...], preferred_element_type=jnp.float32)
    alciod[...] = jnp.dot(h2, aciod[...], preferred_element_type=jnp.float32)


def _final_cell_body(ncc0, ncc1, dcc0, dcc1, nio0, nio1, dio0, dio1,
                     p, bias, wt, bt, g, beta, out):
    pv = p[...]
    rd_cc = jnp.dot(1.0 / (dcc0[...] + dcc1[...] + 1e-16), pv,
                    preferred_element_type=jnp.float32)
    rd_io = jnp.dot(1.0 / (dio0[...] + dio1[...] + 1e-16), pv,
                    preferred_element_type=jnp.float32)
    raw = ((ncc0[...] + ncc1[...]) * rd_cc +
           (nio0[...] + nio1[...]) * rd_io + bias[...])
    y = jnp.dot(raw, wt[...], preferred_element_type=jnp.float32) + bt[...]
    mu = jnp.mean(y, axis=-1, keepdims=True)
    var = jnp.mean((y - mu) ** 2, axis=-1, keepdims=True)
    y = (y - mu) * lax.rsqrt(var + 1e-5) * g[...] + beta[...]
    out[...] = raw + jnp.maximum(y, 0.0)


def _final_io_body(n0, n1, d0, d1, p, bias, wt, bt, g, beta, out):
    rd = jnp.dot(1.0 / (d0[...] + d1[...] + 1e-16), p[...],
                 preferred_element_type=jnp.float32)
    raw = (n0[...] + n1[...]) * rd + bias[...]
    y = jnp.dot(raw, wt[...], preferred_element_type=jnp.float32) + bt[...]
    mu = jnp.mean(y, axis=-1, keepdims=True)
    var = jnp.mean((y - mu) ** 2, axis=-1, keepdims=True)
    y = (y - mu) * lax.rsqrt(var + 1e-5) * g[...] + beta[...]
    out[...] = raw + jnp.maximum(y, 0.0)


def _row_spec(bm, w):
    return pl.BlockSpec((bm, w), lambda i: (i, 0))


def _full_spec(shape):
    nd = len(shape)
    return pl.BlockSpec(shape, lambda i: (0,) * nd)


def _prep_cell(x, wcc, wcio, wioc, accs, accd, acios, aiocd):
    n = NCELL_P
    f32 = jnp.float32
    return pl.pallas_call(
        _prep_cell_body,
        grid=(n // BM,),
        in_specs=[_row_spec(BM, HD)] + [
            _full_spec(w.shape)
            for w in (wcc, wcio, wioc, accs, accd, acios, aiocd)],
        out_specs=[_row_spec(BM, HD), _row_spec(BM, HD), _row_spec(BM, 4),
                   _row_spec(BM, 4), _row_spec(BM, 4), _row_spec(BM, 4)],
        out_shape=[jax.ShapeDtypeStruct((n, HD), f32),
                   jax.ShapeDtypeStruct((n, HD), f32),
                   jax.ShapeDtypeStruct((n, 4), f32),
                   jax.ShapeDtypeStruct((n, 4), f32),
                   jax.ShapeDtypeStruct((n, 4), f32),
                   jax.ShapeDtypeStruct((n, 4), f32)],
    )(x, wcc, wcio, wioc, accs, accd, acios, aiocd)


def _prep_io(x, wioc, wcio, aiocs, aciod):
    n = NIO_P
    f32 = jnp.float32
    return pl.pallas_call(
        _prep_io_body,
        grid=(n // BM,),
        in_specs=[_row_spec(BM, HD)] + [_full_spec(w.shape)
                                        for w in (wioc, wcio, aiocs, aciod)],
        out_specs=[_row_spec(BM, HD), _row_spec(BM, 4), _row_spec(BM, 4)],
        out_shape=[jax.ShapeDtypeStruct((n, HD), f32),
                   jax.ShapeDtypeStruct((n, 4), f32),
                   jax.ShapeDtypeStruct((n, 4), f32)],
    )(x, wioc, wcio, aiocs, aciod)


def _final_cell(ncc0, ncc1, dcc0, dcc1, nio0, nio1, dio0, dio1,
                p, bias, wt, bt, g, beta):
    return pl.pallas_call(
        _final_cell_body,
        grid=(NCELL // BM,),
        in_specs=[_row_spec(BM, HD), _row_spec(BM, HD), _row_spec(BM, 4),
                  _row_spec(BM, 4), _row_spec(BM, HD), _row_spec(BM, HD),
                  _row_spec(BM, 4), _row_spec(BM, 4), _full_spec((4, HD)),
                  _full_spec((1, HD)), _full_spec((HD, HD)),
                  _full_spec((1, HD)), _full_spec((1, HD)),
                  _full_spec((1, HD))],
        out_specs=_row_spec(BM, HD),
        out_shape=jax.ShapeDtypeStruct((NCELL, HD), jnp.float32),
    )(ncc0, ncc1, dcc0, dcc1, nio0, nio1, dio0, dio1, p, bias, wt, bt, g,
      beta)


def _final_io(n0, n1, d0, d1, p, bias, wt, bt, g, beta):
    return pl.pallas_call(
        _final_io_body,
        grid=(NIO // BM,),
        in_specs=[_row_spec(BM, HD), _row_spec(BM, HD), _row_spec(BM, 4),
                  _row_spec(BM, 4), _full_spec((4, HD)), _full_spec((1, HD)),
                  _full_spec((HD, HD)), _full_spec((1, HD)),
                  _full_spec((1, HD)), _full_spec((1, HD))],
        out_specs=_row_spec(BM, HD),
        out_shape=jax.ShapeDtypeStruct((NIO, HD), jnp.float32),
    )(n0, n1, d0, d1, p, bias, wt, bt, g, beta)


TAB_CELL = NCELL_P * 4
TAB_IO = NIO_P * 4


def _edge_pass(e, hcc, hcio, hioc):
    f32 = jnp.float32
    i32 = jnp.int32
    mesh = plsc.VectorSubcoreMesh(core_axis_name="c", subcore_axis_name="s")

    out_type = [
        jax.ShapeDtypeStruct((NC, NUMP_CELL, HD), f32),
        jax.ShapeDtypeStruct((NC, NUMP_CELL * 4), f32),
        jax.ShapeDtypeStruct((NC, NUMP_CELL, HD), f32),
        jax.ShapeDtypeStruct((NC, NUMP_CELL * 4), f32),
        jax.ShapeDtypeStruct((NC, NUMP_IO, HD), f32),
        jax.ShapeDtypeStruct((NC, NUMP_IO * 4), f32),
    ]
    scratch = [
        pltpu.VMEM((BSZ,), i32),
        pltpu.VMEM((BSZ,), i32),
        pltpu.VMEM((CLEN,), i32),
        pltpu.VMEM((CLEN,), i32),
        pltpu.VMEM((128,), i32),
        pltpu.VMEM((512,), i32),
        pltpu.VMEM((512,), i32),
        pltpu.VMEM((512,), i32),
        pltpu.VMEM((512,), f32),
        pltpu.VMEM((512,), f32),
        pltpu.VMEM((512,), f32),
        pltpu.VMEM((128, HD), f32),
        pltpu.VMEM((ACC_R // NS // 2, HD), f32),
        pltpu.VMEM((ACC_R * 4 // NS,), f32),
        pltpu.VMEM((TAB_CELL // NS,), f32),
        pltpu.VMEM_SHARED((TAB_CELL,), f32),
        pltpu.VMEM_SHARED((TAB_CELL,), f32),
        pltpu.VMEM_SHARED((ACC_R * 4,), f32),
        pltpu.VMEM_SHARED((ACC_R, HD), f32),
    ]

    @functools.partial(pl.kernel, out_type=out_type, mesh=mesh,
                       scratch_types=scratch,
                       compiler_params=pltpu.CompilerParams(
                           needs_layout_passes=False))
    def k(scc, dcc, scio, dcio, sioc, dioc,
          fccs, fccd, fcios, fciod, fiocs, fiocd,
          hcc_h, hcio_h, hioc_h,
          np_cc, dp_cc, np_ioc, dp_ioc, np_cio, dp_cio,
          srcb, dstb, csrc, cdst, sdst, sidx, didx, dlidx,
          asb, adb, exb, hbuf, zbuf, zbufd, sbuf, tabs, tabd, den, acc):
        c = lax.axis_index("c")
        s = lax.axis_index("s")
        wid = c * NS + s
        i16 = lax.iota(i32, 16)
        l4 = jnp.bitwise_and(i16, 3)
        e4 = jnp.right_shift(i16, 2)
        zrows = ACC_R // NS // 2
        zdlen = ACC_R * 4 // NS
        z16 = jnp.zeros((16,), f32)
        jk16 = jnp.full((16,), JUNK, i32)
        zi16 = jnp.zeros((16,), i32)

        dnum = lax.GatherDimensionNumbers(offset_dims=(),
                                          collapsed_slice_dims=(0,),
                                          start_index_map=(0,))

        def perm16(x, perm):
            return lax.gather(x, perm.reshape(16, 1), dnum, (1,),
                              mode=lax.GatherScatterMode.PROMISE_IN_BOUNDS)

        def prefix16(x):
            for kk in (1, 2, 4, 8):
                sh = perm16(x, jnp.maximum(i16 - kk, 0))
                x = x + jnp.where(i16 >= kk, sh, 0)
            return x

        def zrow(i, _):
            for q in range(HD // 16):
                zbuf[i, pl.ds(q * 16, 16)] = z16
            return 0
        lax.fori_loop(0, zrows, zrow, 0)

        def zd(i, _):
            o = pl.multiple_of(16 * i, 16)
            zbufd[pl.ds(o, 16)] = z16
            return 0
        lax.fori_loop(0, zdlen // 16, zd, 0)

        def run_phase(src_h, dst_h, ftabs_h, ftabd_h, h_h, nblk, lo, hi,
                      ntab_d, stage_s, stage_d, np_h, dp_h, acc_lim, kk):
            pltpu.sync_copy(ftabs_h.at[pl.ds(s * stage_s, stage_s)],
                            sbuf.at[pl.ds(0, stage_s)])
            pltpu.sync_copy(sbuf.at[pl.ds(0, stage_s)],
                            tabs.at[pl.ds(s * stage_s, stage_s)])
            pltpu.sync_copy(ftabd_h.at[pl.ds(s * stage_d, stage_d)],
                            sbuf.at[pl.ds(0, stage_d)])
            pltpu.sync_copy(sbuf.at[pl.ds(0, stage_d)],
                            tabd.at[pl.ds(s * stage_d, stage_d)])
            pltpu.sync_copy(zbuf, acc.at[pl.ds(s * 2 * zrows, zrows)])
            pltpu.sync_copy(zbuf, acc.at[pl.ds(s * 2 * zrows + zrows, zrows)])
            pltpu.sync_copy(zbufd, den.at[pl.ds(s * zdlen, zdlen)])
            plsc.subcore_barrier()

            ebase = wid * nblk * BSZ

            def blk(b, _):
                off = pl.multiple_of(ebase + b * BSZ, BSZ)
                pltpu.sync_copy(src_h.at[pl.ds(off, BSZ)], srcb)
                pltpu.sync_copy(dst_h.at[pl.ds(off, BSZ)], dstb)

                def grp(g, ncur):
                    o16 = pl.multiple_of(16 * g, 16)
                    dg = dstb[pl.ds(o16, 16)]
                    sg = srcb[pl.ds(o16, 16)]
                    m = jnp.logical_and(dg >= lo, dg < hi)
                    psum = prefix16(jnp.where(m, 1, 0).astype(i32))
                    pos = jnp.where(m, ncur + psum - 1, TRASH)
                    plsc.store_scatter(cdst, [pos], dg - lo)
                    plsc.store_scatter(csrc, [pos], sg)
                    return ncur + jnp.max(psum)
                n = lax.fori_loop(0, BSZ // 16, grp, 0)

                for q in range(8):
                    cdst[pl.ds(n + 16 * q, 16)] = jk16
                    csrc[pl.ds(n + 16 * q, 16)] = zi16

                def bat(j, _):
                    o = pl.multiple_of(j * 128, 128)
                    for q in range(8):
                        oq = o + 16 * q
                        vs = csrc[pl.ds(oq, 16)]
                        vd = cdst[pl.ds(oq, 16)]
                        sdst[pl.ds(16 * q, 16)] = vd
                        vdg = jnp.minimum(vd + lo, ntab_d - 1)
                        for hh in range(4):
                            ii = 64 * q + 16 * hh
                            rs = perm16(vs, e4 + 4 * hh)
                            sidx[pl.ds(ii, 16)] = 4 * rs + l4
                            rdg = perm16(vdg, e4 + 4 * hh)
                            didx[pl.ds(ii, 16)] = 4 * rdg + l4
                            rdl = perm16(vd, e4 + 4 * hh)
                            dlidx[pl.ds(ii, 16)] = 4 * rdl + l4
                    pltpu.sync_copy(tabs.at[sidx], asb)
                    pltpu.sync_copy(tabd.at[didx], adb)

                    def egr(g2, _):
                        o16 = pl.multiple_of(16 * g2, 16)
                        ev = asb[pl.ds(o16, 16)] + adb[pl.ds(o16, 16)]
                        ev = jnp.maximum(ev, 0.2 * ev)
                        exb[pl.ds(o16, 16)] = jnp.exp(ev)
                        return 0
                    lax.fori_loop(0, 32, egr, 0)

                    pltpu.sync_copy(exb, den.at[dlidx], add=True)
                    pltpu.sync_copy(h_h.at[csrc.at[pl.ds(o, 128)]], hbuf)

                    def edge(i, _):
                        for hh in range(4):
                            al = plsc.load_gather(
                                exb, [jnp.full((16,), 4 * i + hh, i32)])
                            for q2 in range(2):
                                col = 32 * hh + 16 * q2
                                hv = hbuf[i, pl.ds(col, 16)]
                                hbuf[i, pl.ds(col, 16)] = hv * al
                        return 0
                    lax.fori_loop(0, 128, edge, 0)

                    pltpu.sync_copy(hbuf, acc.at[sdst], add=True)
                    return 0
                nbat = jnp.right_shift(n + 127, 7)
                lax.fori_loop(0, nbat, bat, 0)
                return 0
            lax.fori_loop(0, nblk, blk, 0)

            plsc.subcore_barrier()
            fr = acc_lim // NS
            pltpu.sync_copy(acc.at[pl.ds(s * fr, fr)],
                            np_h.at[c, pl.ds(kk * acc_lim + s * fr, fr)])
            fd = acc_lim * 4 // NS
            pltpu.sync_copy(den.at[pl.ds(s * fd, fd)],
                            dp_h.at[c, pl.ds(kk * acc_lim * 4 + s * fd, fd)])
            plsc.subcore_barrier()

        def chunk(kk, _):
            lo = kk * CH
            run_phase(scc, dcc, fccs, fccd, hcc_h, NBLK_CC, lo, lo + CH,
                      NCELL_P, TAB_CELL // NS, TAB_CELL // NS,
                      np_cc, dp_cc, CH, kk)
            run_phase(sioc, dioc, fiocs, fiocd, hioc_h, NBLK_IO, lo, lo + CH,
                      NCELL_P, TAB_IO // NS, TAB_CELL // NS,
                      np_ioc, dp_ioc, CH, kk)
            return 0
        lax.fori_loop(0, NCH_CELL, chunk, 0)

        def chunk_io(kk, _):
            lo = kk * CH
            run_phase(scio, dcio, fcios, fciod, hcio_h, NBLK_IO, lo, lo + CH,
                      NIO_P, TAB_CELL // NS, TAB_IO // NS,
                      np_cio, dp_cio, CH, kk)
            return 0
        lax.fori_loop(0, NCH_IO, chunk_io, 0)

    return k(e["scc"], e["dcc"], e["scio"], e["dcio"], e["sioc"], e["dioc"],
             e["fccs"], e["fccd"], e["fcios"], e["fciod"], e["fiocs"],
             e["fiocd"], hcc, hcio, hioc)


def _attn_mat(a):
    ar = jnp.arange(HD)
    return jnp.zeros((HD, 4), jnp.float32).at[ar, ar // C].set(a.reshape(-1))


def _pad_edges(src, dst, ep, npad):
    ne = src.shape[0]
    return (jnp.pad(src, (0, ep - ne)),
            jnp.pad(dst, (0, ep - ne), constant_values=npad))


def kernel(x_cell, x_io, edge_index_cc, edge_index_cio, edge_index_ioc,
           W_cc, a_src_cc, a_dst_cc, b_cc,
           W_cio, a_src_cio, a_dst_cio, b_cio,
           W_ioc, a_src_ioc, a_dst_ioc, b_ioc,
           Wt_cell, bt_cell, g_cell, beta_cell,
           Wt_io, bt_io, g_io, beta_io):
    f32 = jnp.float32
    xc = jnp.pad(x_cell, ((0, NCELL_P - NCELL), (0, 0)))
    xi = jnp.pad(x_io, ((0, NIO_P - NIO), (0, 0)))

    hcc, hcio, alccs, alccd, alcios, aliocd = _prep_cell(
        xc, W_cc, W_cio, W_ioc, _attn_mat(a_src_cc), _attn_mat(a_dst_cc),
        _attn_mat(a_src_cio), _attn_mat(a_dst_ioc))
    hioc, aliocs, alciod = _prep_io(
        xi, W_ioc, W_cio, _attn_mat(a_src_ioc), _attn_mat(a_dst_cio))

    e = {}
    e["scc"], e["dcc"] = _pad_edges(edge_index_cc[0], edge_index_cc[1],
                                    EP_CC, NCELL)
    e["scio"], e["dcio"] = _pad_edges(edge_index_cio[0], edge_index_cio[1],
                                      EP_IO, NIO)
    e["sioc"], e["dioc"] = _pad_edges(edge_index_ioc[0], edge_index_ioc[1],
                                      EP_IO, NCELL)
    e["fccs"] = alccs.reshape(-1)
    e["fccd"] = alccd.reshape(-1)
    e["fcios"] = alcios.reshape(-1)
    e["fciod"] = alciod.reshape(-1)
    e["fiocs"] = aliocs.reshape(-1)
    e["fiocd"] = aliocd.reshape(-1)

    np_cc, dp_cc, np_ioc, dp_ioc, np_cio, dp_cio = _edge_pass(
        e, hcc, hcio, hioc)

    p = (jnp.arange(HD)[None, :] // C == jnp.arange(4)[:, None]).astype(f32)
    dcc = dp_cc.reshape(NC, NUMP_CELL, 4)
    dioc = dp_ioc.reshape(NC, NUMP_CELL, 4)
    dcio = dp_cio.reshape(NC, NUMP_IO, 4)

    cell = _final_cell(
        np_cc[0, :NCELL], np_cc[1, :NCELL], dcc[0, :NCELL], dcc[1, :NCELL],
        np_ioc[0, :NCELL], np_ioc[1, :NCELL], dioc[0, :NCELL],
        dioc[1, :NCELL], p, (b_cc + b_ioc).reshape(1, HD), Wt_cell,
        bt_cell.reshape(1, HD), g_cell.reshape(1, HD),
        beta_cell.reshape(1, HD))
    io = _final_io(
        np_cio[0, :NIO], np_cio[1, :NIO], dcio[0, :NIO], dcio[1, :NIO],
        p, b_cio.reshape(1, HD), Wt_io, bt_io.reshape(1, HD),
        g_io.reshape(1, HD), beta_io.reshape(1, HD))
    return (cell, io)


# speedup vs baseline: 4.8081x; 1.0887x over previous
import functools

import jax
import jax.numpy as jnp
from jax import lax
from jax.experimental import pallas as pl
from jax.experimental.pallas import tpu as pltpu
from jax.experimental.pallas import tpu_sc as plsc

HD = 128
H = 4
C = 32
NCELL = 50000
NIO = 10000

NC = 2
NS = 16
NW = NC * NS

BM = 400
NCELL_P = 50400
NIO_P = 10400

BSZ = 2048
NBLK_CC = 8
EP_CC = NW * NBLK_CC * BSZ
NBLK_IO = 2
EP_IO = NW * NBLK_IO * BSZ
BAT = 128

CH = 4096
NCH_CELL = 13
NUMP_CELL = NCH_CELL * CH
NCH_IO = 3
NUMP_IO = NCH_IO * CH
ACC_R = 4352
JUNK = ACC_R - 1
CLEN = BSZ + 512
TRASH = BSZ + 384


def _prep_cell_body(x, wcc, wcio, wioc, accs, accd, acios, aiocd,
                    hcc, hcio, alccs, alccd, alcios, aliocd):
    xv = x[...]
    h1 = jnp.dot(xv, wcc[...], preferred_element_type=jnp.float32)
    h2 = jnp.dot(xv, wcio[...], preferred_element_type=jnp.float32)
    h3 = jnp.dot(xv, wioc[...], preferred_element_type=jnp.float32)
    hcc[...] = h1
    hcio[...] = h2
    alccs[...] = jnp.dot(h1, accs[...], preferred_element_type=jnp.float32)
    alccd[...] = jnp.dot(h1, accd[...], preferred_element_type=jnp.float32)
    alcios[...] = jnp.dot(h2, acios[...], preferred_element_type=jnp.float32)
    aliocd[...] = jnp.dot(h3, aiocd[...], preferred_element_type=jnp.float32)


def _prep_io_body(x, wioc, wcio, aiocs, aciod, hioc, aliocs, alciod):
    xv = x[...]
    h1 = jnp.dot(xv, wioc[...], preferred_element_type=jnp.float32)
    h2 = jnp.dot(xv, wcio[...], preferred_element_type=jnp.float32)
    hioc[...] = h1
    aliocs[...] = jnp.dot(h1, aiocs[...], preferred_element_type=jnp.float32)
    alciod[...] = jnp.dot(h2, aciod[...], preferred_element_type=jnp.float32)


def _final_cell_body(ncc0, ncc1, dcc0, dcc1, nio0, nio1, dio0, dio1,
                     p, bias, wt, bt, g, beta, out):
    pv = p[...]
    rd_cc = jnp.dot(1.0 / (dcc0[...] + dcc1[...] + 1e-16), pv,
                    preferred_element_type=jnp.float32)
    rd_io = jnp.dot(1.0 / (dio0[...] + dio1[...] + 1e-16), pv,
                    preferred_element_type=jnp.float32)
    raw = ((ncc0[...] + ncc1[...]) * rd_cc +
           (nio0[...] + nio1[...]) * rd_io + bias[...])
    y = jnp.dot(raw, wt[...], preferred_element_type=jnp.float32) + bt[...]
    mu = jnp.mean(y, axis=-1, keepdims=True)
    var = jnp.mean((y - mu) ** 2, axis=-1, keepdims=True)
    y = (y - mu) * lax.rsqrt(var + 1e-5) * g[...] + beta[...]
    out[...] = raw + jnp.maximum(y, 0.0)


def _final_io_body(n0, n1, d0, d1, p, bias, wt, bt, g, beta, out):
    rd = jnp.dot(1.0 / (d0[...] + d1[...] + 1e-16), p[...],
                 preferred_element_type=jnp.float32)
    raw = (n0[...] + n1[...]) * rd + bias[...]
    y = jnp.dot(raw, wt[...], preferred_element_type=jnp.float32) + bt[...]
    mu = jnp.mean(y, axis=-1, keepdims=True)
    var = jnp.mean((y - mu) ** 2, axis=-1, keepdims=True)
    y = (y - mu) * lax.rsqrt(var + 1e-5) * g[...] + beta[...]
    out[...] = raw + jnp.maximum(y, 0.0)


def _row_spec(bm, w):
    return pl.BlockSpec((bm, w), lambda i: (i, 0))


def _full_spec(shape):
    nd = len(shape)
    return pl.BlockSpec(shape, lambda i: (0,) * nd)


def _prep_cell(x, wcc, wcio, wioc, accs, accd, acios, aiocd):
    n = NCELL_P
    f32 = jnp.float32
    return pl.pallas_call(
        _prep_cell_body,
        grid=(n // BM,),
        in_specs=[_row_spec(BM, HD)] + [
            _full_spec(w.shape)
            for w in (wcc, wcio, wioc, accs, accd, acios, aiocd)],
        out_specs=[_row_spec(BM, HD), _row_spec(BM, HD), _row_spec(BM, 4),
                   _row_spec(BM, 4), _row_spec(BM, 4), _row_spec(BM, 4)],
        out_shape=[jax.ShapeDtypeStruct((n, HD), f32),
                   jax.ShapeDtypeStruct((n, HD), f32),
                   jax.ShapeDtypeStruct((n, 4), f32),
                   jax.ShapeDtypeStruct((n, 4), f32),
                   jax.ShapeDtypeStruct((n, 4), f32),
                   jax.ShapeDtypeStruct((n, 4), f32)],
    )(x, wcc, wcio, wioc, accs, accd, acios, aiocd)


def _prep_io(x, wioc, wcio, aiocs, aciod):
    n = NIO_P
    f32 = jnp.float32
    return pl.pallas_call(
        _prep_io_body,
        grid=(n // BM,),
        in_specs=[_row_spec(BM, HD)] + [_full_spec(w.shape)
                                        for w in (wioc, wcio, aiocs, aciod)],
        out_specs=[_row_spec(BM, HD), _row_spec(BM, 4), _row_spec(BM, 4)],
        out_shape=[jax.ShapeDtypeStruct((n, HD), f32),
                   jax.ShapeDtypeStruct((n, 4), f32),
                   jax.ShapeDtypeStruct((n, 4), f32)],
    )(x, wioc, wcio, aiocs, aciod)


def _final_cell(ncc0, ncc1, dcc0, dcc1, nio0, nio1, dio0, dio1,
                p, bias, wt, bt, g, beta):
    return pl.pallas_call(
        _final_cell_body,
        grid=(NCELL // BM,),
        in_specs=[_row_spec(BM, HD), _row_spec(BM, HD), _row_spec(BM, 4),
                  _row_spec(BM, 4), _row_spec(BM, HD), _row_spec(BM, HD),
                  _row_spec(BM, 4), _row_spec(BM, 4), _full_spec((4, HD)),
                  _full_spec((1, HD)), _full_spec((HD, HD)),
                  _full_spec((1, HD)), _full_spec((1, HD)),
                  _full_spec((1, HD))],
        out_specs=_row_spec(BM, HD),
        out_shape=jax.ShapeDtypeStruct((NCELL, HD), jnp.float32),
    )(ncc0, ncc1, dcc0, dcc1, nio0, nio1, dio0, dio1, p, bias, wt, bt, g,
      beta)


def _final_io(n0, n1, d0, d1, p, bias, wt, bt, g, beta):
    return pl.pallas_call(
        _final_io_body,
        grid=(NIO // BM,),
        in_specs=[_row_spec(BM, HD), _row_spec(BM, HD), _row_spec(BM, 4),
                  _row_spec(BM, 4), _full_spec((4, HD)), _full_spec((1, HD)),
                  _full_spec((HD, HD)), _full_spec((1, HD)),
                  _full_spec((1, HD)), _full_spec((1, HD))],
        out_specs=_row_spec(BM, HD),
        out_shape=jax.ShapeDtypeStruct((NIO, HD), jnp.float32),
    )(n0, n1, d0, d1, p, bias, wt, bt, g, beta)


TAB_CELL = NCELL_P * 4
TAB_IO = NIO_P * 4


def _edge_pass(e, hcc, hcio, hioc):
    f32 = jnp.float32
    i32 = jnp.int32
    mesh = plsc.VectorSubcoreMesh(core_axis_name="c", subcore_axis_name="s")

    out_type = [
        jax.ShapeDtypeStruct((NC, NUMP_CELL, HD), f32),
        jax.ShapeDtypeStruct((NC, NUMP_CELL * 4), f32),
        jax.ShapeDtypeStruct((NC, NUMP_CELL, HD), f32),
        jax.ShapeDtypeStruct((NC, NUMP_CELL * 4), f32),
        jax.ShapeDtypeStruct((NC, NUMP_IO, HD), f32),
        jax.ShapeDtypeStruct((NC, NUMP_IO * 4), f32),
    ]
    scratch = [
        pltpu.VMEM((BSZ,), i32),
        pltpu.VMEM((BSZ,), i32),
        pltpu.VMEM((CLEN,), i32),
        pltpu.VMEM((CLEN,), i32),
        pltpu.VMEM((BAT,), i32),
        pltpu.VMEM((BAT * 4,), i32),
        pltpu.VMEM((BAT * 4,), i32),
        pltpu.VMEM((BAT * 4,), i32),
        pltpu.VMEM((BAT * 4,), f32),
        pltpu.VMEM((BAT * 4,), f32),
        pltpu.VMEM((BAT * 4,), f32),
        pltpu.VMEM((BAT, HD), f32),
        pltpu.VMEM((ACC_R // NS // 2, HD), f32),
        pltpu.VMEM((ACC_R * 4 // NS,), f32),
        pltpu.VMEM((TAB_CELL // NS,), f32),
        pltpu.VMEM_SHARED((TAB_CELL,), f32),
        pltpu.VMEM_SHARED((TAB_CELL,), f32),
        pltpu.VMEM_SHARED((ACC_R * 4,), f32),
        pltpu.VMEM_SHARED((ACC_R, HD), f32),
    ]

    @functools.partial(pl.kernel, out_type=out_type, mesh=mesh,
                       scratch_types=scratch,
                       compiler_params=pltpu.CompilerParams(
                           needs_layout_passes=False))
    def k(scc, dcc, scio, dcio, sioc, dioc,
          fccs, fccd, fcios, fciod, fiocs, fiocd,
          hcc_h, hcio_h, hioc_h,
          np_cc, dp_cc, np_ioc, dp_ioc, np_cio, dp_cio,
          srcb, dstb, csrc, cdst, sdst, sidx, didx, dlidx,
          asb, adb, exb, hbuf, zbuf, zbufd, sbuf, tabs, tabd, den, acc):
        c = lax.axis_index("c")
        s = lax.axis_index("s")
        wid = c * NS + s
        i16 = lax.iota(i32, 16)
        l4 = jnp.bitwise_and(i16, 3)
        e4 = jnp.right_shift(i16, 2)
        zrows = ACC_R // NS // 2
        zdlen = ACC_R * 4 // NS
        z16 = jnp.zeros((16,), f32)
        jk16 = jnp.full((16,), JUNK, i32)
        zi16 = jnp.zeros((16,), i32)

        dnum = lax.GatherDimensionNumbers(offset_dims=(),
                                          collapsed_slice_dims=(0,),
                                          start_index_map=(0,))

        def perm16(x, perm):
            return lax.gather(x, perm.reshape(16, 1), dnum, (1,),
                              mode=lax.GatherScatterMode.PROMISE_IN_BOUNDS)

        def prefix16(x):
            for kk in (1, 2, 4, 8):
                sh = perm16(x, jnp.maximum(i16 - kk, 0))
                x = x + jnp.where(i16 >= kk, sh, 0)
            return x

        def zrow(i, _):
            for q in range(HD // 16):
                zbuf[i, pl.ds(q * 16, 16)] = z16
            return 0
        lax.fori_loop(0, zrows, zrow, 0)

        def zd(i, _):
            o = pl.multiple_of(16 * i, 16)
            zbufd[pl.ds(o, 16)] = z16
            return 0
        lax.fori_loop(0, zdlen // 16, zd, 0)

        def run_phase(src_h, dst_h, ftabs_h, ftabd_h, h_h, nblk, lo, hi,
                      ntab_d, stage_s, stage_d, np_h, dp_h, acc_lim, kk):
            pltpu.sync_copy(ftabs_h.at[pl.ds(s * stage_s, stage_s)],
                            sbuf.at[pl.ds(0, stage_s)])
            pltpu.sync_copy(sbuf.at[pl.ds(0, stage_s)],
                            tabs.at[pl.ds(s * stage_s, stage_s)])
            pltpu.sync_copy(ftabd_h.at[pl.ds(s * stage_d, stage_d)],
                            sbuf.at[pl.ds(0, stage_d)])
            pltpu.sync_copy(sbuf.at[pl.ds(0, stage_d)],
                            tabd.at[pl.ds(s * stage_d, stage_d)])
            pltpu.sync_copy(zbuf, acc.at[pl.ds(s * 2 * zrows, zrows)])
            pltpu.sync_copy(zbuf, acc.at[pl.ds(s * 2 * zrows + zrows, zrows)])
            pltpu.sync_copy(zbufd, den.at[pl.ds(s * zdlen, zdlen)])
            plsc.subcore_barrier()

            ebase = wid * nblk * BSZ

            def blk(b, _):
                off = pl.multiple_of(ebase + b * BSZ, BSZ)
                pltpu.sync_copy(src_h.at[pl.ds(off, BSZ)], srcb)
                pltpu.sync_copy(dst_h.at[pl.ds(off, BSZ)], dstb)

                def grp(g, ncur):
                    o16 = pl.multiple_of(16 * g, 16)
                    dg = dstb[pl.ds(o16, 16)]
                    sg = srcb[pl.ds(o16, 16)]
                    m = jnp.logical_and(dg >= lo, dg < hi)
                    psum = prefix16(jnp.where(m, 1, 0).astype(i32))
                    pos = jnp.where(m, ncur + psum - 1, TRASH)
                    plsc.store_scatter(cdst, [pos], dg - lo)
                    plsc.store_scatter(csrc, [pos], sg)
                    return ncur + jnp.max(psum)
                n = lax.fori_loop(0, BSZ // 16, grp, 0)

                for q in range(16):
                    cdst[pl.ds(n + 16 * q, 16)] = jk16
                    csrc[pl.ds(n + 16 * q, 16)] = zi16

                def bat(j, _):
                    o = pl.multiple_of(j * BAT, BAT)
                    for q in range(BAT // 16):
                        oq = o + 16 * q
                        vs = csrc[pl.ds(oq, 16)]
                        vd = cdst[pl.ds(oq, 16)]
                        sdst[pl.ds(16 * q, 16)] = vd
                        vdg = jnp.minimum(vd + lo, ntab_d - 1)
                        for hh in range(4):
                            ii = 64 * q + 16 * hh
                            rs = perm16(vs, e4 + 4 * hh)
                            sidx[pl.ds(ii, 16)] = 4 * rs + l4
                            rdg = perm16(vdg, e4 + 4 * hh)
                            didx[pl.ds(ii, 16)] = 4 * rdg + l4
                            rdl = perm16(vd, e4 + 4 * hh)
                            dlidx[pl.ds(ii, 16)] = 4 * rdl + l4
                    pltpu.sync_copy(tabs.at[sidx], asb)
                    pltpu.sync_copy(tabd.at[didx], adb)

                    def egr(g2, _):
                        o16 = pl.multiple_of(16 * g2, 16)
                        ev = asb[pl.ds(o16, 16)] + adb[pl.ds(o16, 16)]
                        ev = jnp.maximum(ev, 0.2 * ev)
                        exb[pl.ds(o16, 16)] = jnp.exp(ev)
                        return 0
                    lax.fori_loop(0, BAT // 4, egr, 0)

                    pltpu.sync_copy(exb, den.at[dlidx], add=True)
                    pltpu.sync_copy(h_h.at[csrc.at[pl.ds(o, BAT)]], hbuf)

                    def edge(g3, _):
                        o16 = pl.multiple_of(16 * g3, 16)
                        al16 = exb[pl.ds(o16, 16)]
                        for jj in range(4):
                            i = 4 * g3 + jj
                            for hh in range(4):
                                alv = perm16(al16,
                                             jnp.full((16,), 4 * jj + hh,
                                                      i32))
                                for q2 in range(2):
                                    col = 32 * hh + 16 * q2
                                    hv = hbuf[i, pl.ds(col, 16)]
                                    hbuf[i, pl.ds(col, 16)] = hv * alv
                        return 0
                    lax.fori_loop(0, BAT // 4, edge, 0)

                    pltpu.sync_copy(hbuf, acc.at[sdst], add=True)
                    return 0
                nbat = jnp.right_shift(n + BAT - 1, BAT.bit_length() - 1)
                lax.fori_loop(0, nbat, bat, 0)
                return 0
            lax.fori_loop(0, nblk, blk, 0)

            plsc.subcore_barrier()
            fr = acc_lim // NS
            pltpu.sync_copy(acc.at[pl.ds(s * fr, fr)],
                            np_h.at[c, pl.ds(kk * acc_lim + s * fr, fr)])
            fd = acc_lim * 4 // NS
            pltpu.sync_copy(den.at[pl.ds(s * fd, fd)],
                            dp_h.at[c, pl.ds(kk * acc_lim * 4 + s * fd, fd)])
            plsc.subcore_barrier()

        def chunk(kk, _):
            lo = kk * CH
            run_phase(scc, dcc, fccs, fccd, hcc_h, NBLK_CC, lo, lo + CH,
                      NCELL_P, TAB_CELL // NS, TAB_CELL // NS,
                      np_cc, dp_cc, CH, kk)
            run_phase(sioc, dioc, fiocs, fiocd, hioc_h, NBLK_IO, lo, lo + CH,
                      NCELL_P, TAB_IO // NS, TAB_CELL // NS,
                      np_ioc, dp_ioc, CH, kk)
            return 0
        lax.fori_loop(0, NCH_CELL, chunk, 0)

        def chunk_io(kk, _):
            lo = kk * CH
            run_phase(scio, dcio, fcios, fciod, hcio_h, NBLK_IO, lo, lo + CH,
                      NIO_P, TAB_CELL // NS, TAB_IO // NS,
                      np_cio, dp_cio, CH, kk)
            return 0
        lax.fori_loop(0, NCH_IO, chunk_io, 0)

    return k(e["scc"], e["dcc"], e["scio"], e["dcio"], e["sioc"], e["dioc"],
             e["fccs"], e["fccd"], e["fcios"], e["fciod"], e["fiocs"],
             e["fiocd"], hcc, hcio, hioc)


def _attn_mat(a):
    ar = jnp.arange(HD)
    return jnp.zeros((HD, 4), jnp.float32).at[ar, ar // C].set(a.reshape(-1))


def _pad_edges(src, dst, ep, npad):
    ne = src.shape[0]
    return (jnp.pad(src, (0, ep - ne)),
            jnp.pad(dst, (0, ep - ne), constant_values=npad))


def kernel(x_cell, x_io, edge_index_cc, edge_index_cio, edge_index_ioc,
           W_cc, a_src_cc, a_dst_cc, b_cc,
           W_cio, a_src_cio, a_dst_cio, b_cio,
           W_ioc, a_src_ioc, a_dst_ioc, b_ioc,
           Wt_cell, bt_cell, g_cell, beta_cell,
           Wt_io, bt_io, g_io, beta_io):
    f32 = jnp.float32
    xc = jnp.pad(x_cell, ((0, NCELL_P - NCELL), (0, 0)))
    xi = jnp.pad(x_io, ((0, NIO_P - NIO), (0, 0)))

    hcc, hcio, alccs, alccd, alcios, aliocd = _prep_cell(
        xc, W_cc, W_cio, W_ioc, _attn_mat(a_src_cc), _attn_mat(a_dst_cc),
        _attn_mat(a_src_cio), _attn_mat(a_dst_ioc))
    hioc, aliocs, alciod = _prep_io(
        xi, W_ioc, W_cio, _attn_mat(a_src_ioc), _attn_mat(a_dst_cio))

    e = {}
    e["scc"], e["dcc"] = _pad_edges(edge_index_cc[0], edge_index_cc[1],
                                    EP_CC, NCELL)
    e["scio"], e["dcio"] = _pad_edges(edge_index_cio[0], edge_index_cio[1],
                                      EP_IO, NIO)
    e["sioc"], e["dioc"] = _pad_edges(edge_index_ioc[0], edge_index_ioc[1],
                                      EP_IO, NCELL)
    e["fccs"] = alccs.reshape(-1)
    e["fccd"] = alccd.reshape(-1)
    e["fcios"] = alcios.reshape(-1)
    e["fciod"] = alciod.reshape(-1)
    e["fiocs"] = aliocs.reshape(-1)
    e["fiocd"] = aliocd.reshape(-1)

    np_cc, dp_cc, np_ioc, dp_ioc, np_cio, dp_cio = _edge_pass(
        e, hcc, hcio, hioc)

    p = (jnp.arange(HD)[None, :] // C == jnp.arange(4)[:, None]).astype(f32)
    dcc = dp_cc.reshape(NC, NUMP_CELL, 4)
    dioc = dp_ioc.reshape(NC, NUMP_CELL, 4)
    dcio = dp_cio.reshape(NC, NUMP_IO, 4)

    cell = _final_cell(
        np_cc[0, :NCELL], np_cc[1, :NCELL], dcc[0, :NCELL], dcc[1, :NCELL],
        np_ioc[0, :NCELL], np_ioc[1, :NCELL], dioc[0, :NCELL],
        dioc[1, :NCELL], p, (b_cc + b_ioc).reshape(1, HD), Wt_cell,
        bt_cell.reshape(1, HD), g_cell.reshape(1, HD),
        beta_cell.reshape(1, HD))
    io = _final_io(
        np_cio[0, :NIO], np_cio[1, :NIO], dcio[0, :NIO], dcio[1, :NIO],
        p, b_cio.reshape(1, HD), Wt_io, bt_io.reshape(1, HD),
        g_io.reshape(1, HD), beta_io.reshape(1, HD))
    return (cell, io)


# speedup vs baseline: 4.8568x; 1.0101x over previous
import functools

import jax
import jax.numpy as jnp
from jax import lax
from jax.experimental import pallas as pl
from jax.experimental.pallas import tpu as pltpu
from jax.experimental.pallas import tpu_sc as plsc

HD = 128
H = 4
C = 32
NCELL = 50000
NIO = 10000

NC = 2
NS = 16
NW = NC * NS

BM = 400
NCELL_P = 50400
NIO_P = 10400

BSZ = 2048
NBLK_CC = 8
EP_CC = NW * NBLK_CC * BSZ
NBLK_IO = 2
EP_IO = NW * NBLK_IO * BSZ
BAT = 128

CH = 4096
NCH_CELL = 13
NUMP_CELL = NCH_CELL * CH
NCH_IO = 3
NUMP_IO = NCH_IO * CH
ACC_R = 4352
JUNK = ACC_R - 1
CLEN = BSZ + 512
TRASH = BSZ + 384


def _prep_cell_body(x, wcc, wcio, wioc, accs, accd, acios, aiocd,
                    hcc, hcio, alccs, alccd, alcios, aliocd):
    xv = x[...]
    h1 = jnp.dot(xv, wcc[...], preferred_element_type=jnp.float32)
    h2 = jnp.dot(xv, wcio[...], preferred_element_type=jnp.float32)
    h3 = jnp.dot(xv, wioc[...], preferred_element_type=jnp.float32)
    hcc[...] = h1
    hcio[...] = h2
    alccs[...] = jnp.dot(h1, accs[...], preferred_element_type=jnp.float32)
    alccd[...] = jnp.dot(h1, accd[...], preferred_element_type=jnp.float32)
    alcios[...] = jnp.dot(h2, acios[...], preferred_element_type=jnp.float32)
    aliocd[...] = jnp.dot(h3, aiocd[...], preferred_element_type=jnp.float32)


def _prep_io_body(x, wioc, wcio, aiocs, aciod, hioc, aliocs, alciod):
    xv = x[...]
    h1 = jnp.dot(xv, wioc[...], preferred_element_type=jnp.float32)
    h2 = jnp.dot(xv, wcio[...], preferred_element_type=jnp.float32)
    hioc[...] = h1
    aliocs[...] = jnp.dot(h1, aiocs[...], preferred_element_type=jnp.float32)
    alciod[...] = jnp.dot(h2, aciod[...], preferred_element_type=jnp.float32)


def _final_cell_body(ncc0, ncc1, dcc0, dcc1, nio0, nio1, dio0, dio1,
                     p, bias, wt, bt, g, beta, out):
    pv = p[...]
    rd_cc = jnp.dot(1.0 / (dcc0[...] + dcc1[...] + 1e-16), pv,
                    preferred_element_type=jnp.float32)
    rd_io = jnp.dot(1.0 / (dio0[...] + dio1[...] + 1e-16), pv,
                    preferred_element_type=jnp.float32)
    raw = ((ncc0[...] + ncc1[...]) * rd_cc +
           (nio0[...] + nio1[...]) * rd_io + bias[...])
    y = jnp.dot(raw, wt[...], preferred_element_type=jnp.float32) + bt[...]
    mu = jnp.mean(y, axis=-1, keepdims=True)
    var = jnp.mean((y - mu) ** 2, axis=-1, keepdims=True)
    y = (y - mu) * lax.rsqrt(var + 1e-5) * g[...] + beta[...]
    out[...] = raw + jnp.maximum(y, 0.0)


def _final_io_body(n0, n1, d0, d1, p, bias, wt, bt, g, beta, out):
    rd = jnp.dot(1.0 / (d0[...] + d1[...] + 1e-16), p[...],
                 preferred_element_type=jnp.float32)
    raw = (n0[...] + n1[...]) * rd + bias[...]
    y = jnp.dot(raw, wt[...], preferred_element_type=jnp.float32) + bt[...]
    mu = jnp.mean(y, axis=-1, keepdims=True)
    var = jnp.mean((y - mu) ** 2, axis=-1, keepdims=True)
    y = (y - mu) * lax.rsqrt(var + 1e-5) * g[...] + beta[...]
    out[...] = raw + jnp.maximum(y, 0.0)


def _row_spec(bm, w):
    return pl.BlockSpec((bm, w), lambda i: (i, 0))


def _full_spec(shape):
    nd = len(shape)
    return pl.BlockSpec(shape, lambda i: (0,) * nd)


def _prep_cell(x, wcc, wcio, wioc, accs, accd, acios, aiocd):
    n = NCELL_P
    f32 = jnp.float32
    return pl.pallas_call(
        _prep_cell_body,
        grid=(n // BM,),
        in_specs=[_row_spec(BM, HD)] + [
            _full_spec(w.shape)
            for w in (wcc, wcio, wioc, accs, accd, acios, aiocd)],
        out_specs=[_row_spec(BM, HD), _row_spec(BM, HD), _row_spec(BM, 4),
                   _row_spec(BM, 4), _row_spec(BM, 4), _row_spec(BM, 4)],
        out_shape=[jax.ShapeDtypeStruct((n, HD), f32),
                   jax.ShapeDtypeStruct((n, HD), f32),
                   jax.ShapeDtypeStruct((n, 4), f32),
                   jax.ShapeDtypeStruct((n, 4), f32),
                   jax.ShapeDtypeStruct((n, 4), f32),
                   jax.ShapeDtypeStruct((n, 4), f32)],
    )(x, wcc, wcio, wioc, accs, accd, acios, aiocd)


def _prep_io(x, wioc, wcio, aiocs, aciod):
    n = NIO_P
    f32 = jnp.float32
    return pl.pallas_call(
        _prep_io_body,
        grid=(n // BM,),
        in_specs=[_row_spec(BM, HD)] + [_full_spec(w.shape)
                                        for w in (wioc, wcio, aiocs, aciod)],
        out_specs=[_row_spec(BM, HD), _row_spec(BM, 4), _row_spec(BM, 4)],
        out_shape=[jax.ShapeDtypeStruct((n, HD), f32),
                   jax.ShapeDtypeStruct((n, 4), f32),
                   jax.ShapeDtypeStruct((n, 4), f32)],
    )(x, wioc, wcio, aiocs, aciod)


def _final_cell(ncc0, ncc1, dcc0, dcc1, nio0, nio1, dio0, dio1,
                p, bias, wt, bt, g, beta):
    return pl.pallas_call(
        _final_cell_body,
        grid=(NCELL // BM,),
        in_specs=[_row_spec(BM, HD), _row_spec(BM, HD), _row_spec(BM, 4),
                  _row_spec(BM, 4), _row_spec(BM, HD), _row_spec(BM, HD),
                  _row_spec(BM, 4), _row_spec(BM, 4), _full_spec((4, HD)),
                  _full_spec((1, HD)), _full_spec((HD, HD)),
                  _full_spec((1, HD)), _full_spec((1, HD)),
                  _full_spec((1, HD))],
        out_specs=_row_spec(BM, HD),
        out_shape=jax.ShapeDtypeStruct((NCELL, HD), jnp.float32),
    )(ncc0, ncc1, dcc0, dcc1, nio0, nio1, dio0, dio1, p, bias, wt, bt, g,
      beta)


def _final_io(n0, n1, d0, d1, p, bias, wt, bt, g, beta):
    return pl.pallas_call(
        _final_io_body,
        grid=(NIO // BM,),
        in_specs=[_row_spec(BM, HD), _row_spec(BM, HD), _row_spec(BM, 4),
                  _row_spec(BM, 4), _full_spec((4, HD)), _full_spec((1, HD)),
                  _full_spec((HD, HD)), _full_spec((1, HD)),
                  _full_spec((1, HD)), _full_spec((1, HD))],
        out_specs=_row_spec(BM, HD),
        out_shape=jax.ShapeDtypeStruct((NIO, HD), jnp.float32),
    )(n0, n1, d0, d1, p, bias, wt, bt, g, beta)


TAB_CELL = NCELL_P * 4
TAB_IO = NIO_P * 4


def _edge_pass(e, hcc, hcio, hioc):
    f32 = jnp.float32
    i32 = jnp.int32
    mesh = plsc.VectorSubcoreMesh(core_axis_name="c", subcore_axis_name="s")

    out_type = [
        jax.ShapeDtypeStruct((NC, NUMP_CELL, HD), f32),
        jax.ShapeDtypeStruct((NC, NUMP_CELL * 4), f32),
        jax.ShapeDtypeStruct((NC, NUMP_CELL, HD), f32),
        jax.ShapeDtypeStruct((NC, NUMP_CELL * 4), f32),
        jax.ShapeDtypeStruct((NC, NUMP_IO, HD), f32),
        jax.ShapeDtypeStruct((NC, NUMP_IO * 4), f32),
    ]
    scratch = [
        pltpu.VMEM((BSZ,), i32),
        pltpu.VMEM((BSZ,), i32),
        pltpu.VMEM((CLEN,), i32),
        pltpu.VMEM((CLEN,), i32),
        pltpu.VMEM((BAT,), i32),
        pltpu.VMEM((BAT * 4,), i32),
        pltpu.VMEM((BAT * 4,), i32),
        pltpu.VMEM((BAT * 4,), i32),
        pltpu.VMEM((BAT * 4,), f32),
        pltpu.VMEM((BAT * 4,), f32),
        pltpu.VMEM((BAT * 4,), f32),
        pltpu.VMEM((BAT, HD), f32),
        pltpu.VMEM((ACC_R // NS // 2, HD), f32),
        pltpu.VMEM((ACC_R * 4 // NS,), f32),
        pltpu.VMEM((TAB_CELL // NS,), f32),
        pltpu.VMEM_SHARED((TAB_CELL,), f32),
        pltpu.VMEM_SHARED((TAB_CELL,), f32),
        pltpu.VMEM_SHARED((ACC_R * 4,), f32),
        pltpu.VMEM_SHARED((ACC_R, HD), f32),
        pltpu.SemaphoreType.DMA,
    ]

    @functools.partial(pl.kernel, out_type=out_type, mesh=mesh,
                       scratch_types=scratch,
                       compiler_params=pltpu.CompilerParams(
                           needs_layout_passes=False))
    def k(scc, dcc, scio, dcio, sioc, dioc,
          fccs, fccd, fcios, fciod, fiocs, fiocd,
          hcc_h, hcio_h, hioc_h,
          np_cc, dp_cc, np_ioc, dp_ioc, np_cio, dp_cio,
          srcb, dstb, csrc, cdst, sdst, sidx, didx, dlidx,
          asb, adb, exb, hbuf, zbuf, zbufd, sbuf, tabs, tabd, den, acc,
          hsem):
        c = lax.axis_index("c")
        s = lax.axis_index("s")
        wid = c * NS + s
        i16 = lax.iota(i32, 16)
        l4 = jnp.bitwise_and(i16, 3)
        e4 = jnp.right_shift(i16, 2)
        zrows = ACC_R // NS // 2
        zdlen = ACC_R * 4 // NS
        z16 = jnp.zeros((16,), f32)
        jk16 = jnp.full((16,), JUNK, i32)
        zi16 = jnp.zeros((16,), i32)

        dnum = lax.GatherDimensionNumbers(offset_dims=(),
                                          collapsed_slice_dims=(0,),
                                          start_index_map=(0,))

        def perm16(x, perm):
            return lax.gather(x, perm.reshape(16, 1), dnum, (1,),
                              mode=lax.GatherScatterMode.PROMISE_IN_BOUNDS)

        def prefix16(x):
            for kk in (1, 2, 4, 8):
                sh = perm16(x, jnp.maximum(i16 - kk, 0))
                x = x + jnp.where(i16 >= kk, sh, 0)
            return x

        def zrow(i, _):
            for q in range(HD // 16):
                zbuf[i, pl.ds(q * 16, 16)] = z16
            return 0
        lax.fori_loop(0, zrows, zrow, 0)

        def zd(i, _):
            o = pl.multiple_of(16 * i, 16)
            zbufd[pl.ds(o, 16)] = z16
            return 0
        lax.fori_loop(0, zdlen // 16, zd, 0)

        def run_phase(src_h, dst_h, ftabs_h, ftabd_h, h_h, nblk, lo, hi,
                      ntab_d, stage_s, stage_d, np_h, dp_h, acc_lim, kk):
            pltpu.sync_copy(ftabs_h.at[pl.ds(s * stage_s, stage_s)],
                            sbuf.at[pl.ds(0, stage_s)])
            pltpu.sync_copy(sbuf.at[pl.ds(0, stage_s)],
                            tabs.at[pl.ds(s * stage_s, stage_s)])
            pltpu.sync_copy(ftabd_h.at[pl.ds(s * stage_d, stage_d)],
                            sbuf.at[pl.ds(0, stage_d)])
            pltpu.sync_copy(sbuf.at[pl.ds(0, stage_d)],
                            tabd.at[pl.ds(s * stage_d, stage_d)])
            pltpu.sync_copy(zbuf, acc.at[pl.ds(s * 2 * zrows, zrows)])
            pltpu.sync_copy(zbuf, acc.at[pl.ds(s * 2 * zrows + zrows, zrows)])
            pltpu.sync_copy(zbufd, den.at[pl.ds(s * zdlen, zdlen)])
            plsc.subcore_barrier()

            ebase = wid * nblk * BSZ

            def blk(b, _):
                off = pl.multiple_of(ebase + b * BSZ, BSZ)
                pltpu.sync_copy(src_h.at[pl.ds(off, BSZ)], srcb)
                pltpu.sync_copy(dst_h.at[pl.ds(off, BSZ)], dstb)

                def grp(g, ncur):
                    o16 = pl.multiple_of(16 * g, 16)
                    dg = dstb[pl.ds(o16, 16)]
                    sg = srcb[pl.ds(o16, 16)]
                    m = jnp.logical_and(dg >= lo, dg < hi)
                    psum = prefix16(jnp.where(m, 1, 0).astype(i32))
                    pos = jnp.where(m, ncur + psum - 1, TRASH)
                    plsc.store_scatter(cdst, [pos], dg - lo)
                    plsc.store_scatter(csrc, [pos], sg)
                    return ncur + jnp.max(psum)
                n = lax.fori_loop(0, BSZ // 16, grp, 0)

                for q in range(16):
                    cdst[pl.ds(n + 16 * q, 16)] = jk16
                    csrc[pl.ds(n + 16 * q, 16)] = zi16

                def bat(j, _):
                    o = pl.multiple_of(j * BAT, BAT)
                    hd = pltpu.async_copy(h_h.at[csrc.at[pl.ds(o, BAT)]],
                                          hbuf, hsem)
                    for q in range(BAT // 16):
                        oq = o + 16 * q
                        vs = csrc[pl.ds(oq, 16)]
                        vd = cdst[pl.ds(oq, 16)]
                        sdst[pl.ds(16 * q, 16)] = vd
                        vdg = jnp.minimum(vd + lo, ntab_d - 1)
                        for hh in range(4):
                            ii = 64 * q + 16 * hh
                            rs = perm16(vs, e4 + 4 * hh)
                            sidx[pl.ds(ii, 16)] = 4 * rs + l4
                            rdg = perm16(vdg, e4 + 4 * hh)
                            didx[pl.ds(ii, 16)] = 4 * rdg + l4
                            rdl = perm16(vd, e4 + 4 * hh)
                            dlidx[pl.ds(ii, 16)] = 4 * rdl + l4
                    pltpu.sync_copy(tabs.at[sidx], asb)
                    pltpu.sync_copy(tabd.at[didx], adb)

                    def egr(g2, _):
                        o16 = pl.multiple_of(16 * g2, 16)
                        ev = asb[pl.ds(o16, 16)] + adb[pl.ds(o16, 16)]
                        ev = jnp.maximum(ev, 0.2 * ev)
                        exb[pl.ds(o16, 16)] = jnp.exp(ev)
                        return 0
                    lax.fori_loop(0, BAT // 4, egr, 0)

                    pltpu.sync_copy(exb, den.at[dlidx], add=True)
                    hd.wait()

                    def edge(g3, _):
                        o16 = pl.multiple_of(16 * g3, 16)
                        al16 = exb[pl.ds(o16, 16)]
                        for jj in range(4):
                            i = 4 * g3 + jj
                            for hh in range(4):
                                alv = perm16(al16,
                                             jnp.full((16,), 4 * jj + hh,
                                                      i32))
                                for q2 in range(2):
                                    col = 32 * hh + 16 * q2
                                    hv = hbuf[i, pl.ds(col, 16)]
                                    hbuf[i, pl.ds(col, 16)] = hv * alv
                        return 0
                    lax.fori_loop(0, BAT // 4, edge, 0)

                    pltpu.sync_copy(hbuf, acc.at[sdst], add=True)
                    return 0
                nbat = jnp.right_shift(n + BAT - 1, BAT.bit_length() - 1)
                lax.fori_loop(0, nbat, bat, 0)
                return 0
            lax.fori_loop(0, nblk, blk, 0)

            plsc.subcore_barrier()
            fr = acc_lim // NS
            pltpu.sync_copy(acc.at[pl.ds(s * fr, fr)],
                            np_h.at[c, pl.ds(kk * acc_lim + s * fr, fr)])
            fd = acc_lim * 4 // NS
            pltpu.sync_copy(den.at[pl.ds(s * fd, fd)],
                            dp_h.at[c, pl.ds(kk * acc_lim * 4 + s * fd, fd)])
            plsc.subcore_barrier()

        def chunk(kk, _):
            lo = kk * CH
            run_phase(scc, dcc, fccs, fccd, hcc_h, NBLK_CC, lo, lo + CH,
                      NCELL_P, TAB_CELL // NS, TAB_CELL // NS,
                      np_cc, dp_cc, CH, kk)
            run_phase(sioc, dioc, fiocs, fiocd, hioc_h, NBLK_IO, lo, lo + CH,
                      NCELL_P, TAB_IO // NS, TAB_CELL // NS,
                      np_ioc, dp_ioc, CH, kk)
            return 0
        lax.fori_loop(0, NCH_CELL, chunk, 0)

        def chunk_io(kk, _):
            lo = kk * CH
            run_phase(scio, dcio, fcios, fciod, hcio_h, NBLK_IO, lo, lo + CH,
                      NIO_P, TAB_CELL // NS, TAB_IO // NS,
                      np_cio, dp_cio, CH, kk)
            return 0
        lax.fori_loop(0, NCH_IO, chunk_io, 0)

    return k(e["scc"], e["dcc"], e["scio"], e["dcio"], e["sioc"], e["dioc"],
             e["fccs"], e["fccd"], e["fcios"], e["fciod"], e["fiocs"],
             e["fiocd"], hcc, hcio, hioc)


def _attn_mat(a):
    ar = jnp.arange(HD)
    return jnp.zeros((HD, 4), jnp.float32).at[ar, ar // C].set(a.reshape(-1))


def _pad_edges(src, dst, ep, npad):
    ne = src.shape[0]
    return (jnp.pad(src, (0, ep - ne)),
            jnp.pad(dst, (0, ep - ne), constant_values=npad))


def kernel(x_cell, x_io, edge_index_cc, edge_index_cio, edge_index_ioc,
           W_cc, a_src_cc, a_dst_cc, b_cc,
           W_cio, a_src_cio, a_dst_cio, b_cio,
           W_ioc, a_src_ioc, a_dst_ioc, b_ioc,
           Wt_cell, bt_cell, g_cell, beta_cell,
           Wt_io, bt_io, g_io, beta_io):
    f32 = jnp.float32
    xc = jnp.pad(x_cell, ((0, NCELL_P - NCELL), (0, 0)))
    xi = jnp.pad(x_io, ((0, NIO_P - NIO), (0, 0)))

    hcc, hcio, alccs, alccd, alcios, aliocd = _prep_cell(
        xc, W_cc, W_cio, W_ioc, _attn_mat(a_src_cc), _attn_mat(a_dst_cc),
        _attn_mat(a_src_cio), _attn_mat(a_dst_ioc))
    hioc, aliocs, alciod = _prep_io(
        xi, W_ioc, W_cio, _attn_mat(a_src_ioc), _attn_mat(a_dst_cio))

    e = {}
    e["scc"], e["dcc"] = _pad_edges(edge_index_cc[0], edge_index_cc[1],
                                    EP_CC, NCELL)
    e["scio"], e["dcio"] = _pad_edges(edge_index_cio[0], edge_index_cio[1],
                                      EP_IO, NIO)
    e["sioc"], e["dioc"] = _pad_edges(edge_index_ioc[0], edge_index_ioc[1],
                                      EP_IO, NCELL)
    e["fccs"] = alccs.reshape(-1)
    e["fccd"] = alccd.reshape(-1)
    e["fcios"] = alcios.reshape(-1)
    e["fciod"] = alciod.reshape(-1)
    e["fiocs"] = aliocs.reshape(-1)
    e["fiocd"] = aliocd.reshape(-1)

    np_cc, dp_cc, np_ioc, dp_ioc, np_cio, dp_cio = _edge_pass(
        e, hcc, hcio, hioc)

    p = (jnp.arange(HD)[None, :] // C == jnp.arange(4)[:, None]).astype(f32)
    dcc = dp_cc.reshape(NC, NUMP_CELL, 4)
    dioc = dp_ioc.reshape(NC, NUMP_CELL, 4)
    dcio = dp_cio.reshape(NC, NUMP_IO, 4)

    cell = _final_cell(
        np_cc[0, :NCELL], np_cc[1, :NCELL], dcc[0, :NCELL], dcc[1, :NCELL],
        np_ioc[0, :NCELL], np_ioc[1, :NCELL], dioc[0, :NCELL],
        dioc[1, :NCELL], p, (b_cc + b_ioc).reshape(1, HD), Wt_cell,
        bt_cell.reshape(1, HD), g_cell.reshape(1, HD),
        beta_cell.reshape(1, HD))
    io = _final_io(
        np_cio[0, :NIO], np_cio[1, :NIO], dcio[0, :NIO], dcio[1, :NIO],
        p, b_cio.reshape(1, HD), Wt_io, bt_io.reshape(1, HD),
        g_io.reshape(1, HD), beta_io.reshape(1, HD))
    return (cell, io)
